# baseline (speedup 1.0000x reference)
import functools

import jax
import jax.numpy as jnp
from jax import lax
from jax.experimental import pallas as pl
from jax.experimental.pallas import tpu as pltpu
from jax.experimental.pallas import tpu_sc as plsc

L = 16
NW = 32
B, N, M = 4, 4096, 4096
W_PER_B = NW // B
SC_CHUNK = 64
SC_TAIL = SC_CHUNK * W_PER_B
N_TC = N - SC_TAIL
SRCS = 8
M_VECS = M // L
TCN = 512
NB = N_TC // TCN
MT = 1024


def _sc_chamfer(sx, sy, sz, tx, ty, tz, w):
    mesh = plsc.VectorSubcoreMesh(core_axis_name="c", subcore_axis_name="s")

    @functools.partial(
        pl.kernel,
        mesh=mesh,
        out_type=jax.ShapeDtypeStruct((NW, L), jnp.float32),
        scratch_types=[
            pltpu.VMEM((SC_CHUNK,), jnp.float32),
            pltpu.VMEM((SC_CHUNK,), jnp.float32),
            pltpu.VMEM((SC_CHUNK,), jnp.float32),
            pltpu.VMEM((SC_CHUNK,), jnp.float32),
            pltpu.VMEM((M,), jnp.float32),
            pltpu.VMEM((M,), jnp.float32),
            pltpu.VMEM((M,), jnp.float32),
            pltpu.VMEM((M,), jnp.float32),
            pltpu.VMEM((L,), jnp.float32),
        ],
    )
    def k(sx_h, sy_h, sz_h, tx_h, ty_h, tz_h, w_h, out_h,
          sx_s, sy_s, sz_s, w_s, tx_s, ty_s, tz_s, t2_s, o_s):
        wid = lax.axis_index("s") * 2 + lax.axis_index("c")
        b = wid // W_PER_B
        src_base = b * N + N_TC + (wid % W_PER_B) * SC_CHUNK
        tgt_base = b * M

        pltpu.sync_copy(sx_h.at[pl.ds(src_base, SC_CHUNK)], sx_s)
        pltpu.sync_copy(sy_h.at[pl.ds(src_base, SC_CHUNK)], sy_s)
        pltpu.sync_copy(sz_h.at[pl.ds(src_base, SC_CHUNK)], sz_s)
        pltpu.sync_copy(w_h.at[pl.ds(src_base, SC_CHUNK)], w_s)
        pltpu.sync_copy(tx_h.at[pl.ds(tgt_base, M)], tx_s)
        pltpu.sync_copy(ty_h.at[pl.ds(tgt_base, M)], ty_s)
        pltpu.sync_copy(tz_h.at[pl.ds(tgt_base, M)], tz_s)

        def bq(v):
            u = lax.bitcast_convert_type(v, jnp.uint32)
            r = ((u + ((u >> 16) & jnp.uint32(1)) + jnp.uint32(0x7FFF))
                 & jnp.uint32(0xFFFF0000))
            return lax.bitcast_convert_type(r, jnp.float32)

        lane = lax.broadcasted_iota(jnp.int32, (L,), 0)

        def t2_body(i, _):
            txv = tx_s[pl.ds(i * L, L)]
            tyv = ty_s[pl.ds(i * L, L)]
            tzv = tz_s[pl.ds(i * L, L)]
            t2_s[pl.ds(i * L, L)] = txv * txv + tyv * tyv + tzv * tzv
            tx_s[pl.ds(i * L, L)] = bq(txv)
            ty_s[pl.ds(i * L, L)] = bq(tyv)
            tz_s[pl.ds(i * L, L)] = bq(tzv)
            return 0
        lax.fori_loop(0, M_VECS, t2_body, 0)

        inf = jnp.float32(jnp.inf)

        def hmin(v):
            for sh in (8, 4, 2, 1):
                v = jnp.minimum(
                    v, v.at[lane ^ sh].get(mode="promise_in_bounds"))
            return v[0]

        def pass_body(p, total):
            base = p * L
            sxv = sx_s[pl.ds(base, L)]
            syv = sy_s[pl.ds(base, L)]
            szv = sz_s[pl.ds(base, L)]
            wv = w_s[pl.ds(base, L)]
            s2v = sxv * sxv + syv * syv + szv * szv
            ws2v = wv * s2v
            sxq = bq(sxv)
            syq = bq(syv)
            szq = bq(szv)

            for half in range(L // SRCS):
                idxs = [half * SRCS + j for j in range(SRCS)]
                bx = [jnp.broadcast_to(-2.0 * sxq[i], (L,)) for i in idxs]
                by = [jnp.broadcast_to(-2.0 * syq[i], (L,)) for i in idxs]
                bz = [jnp.broadcast_to(-2.0 * szq[i], (L,)) for i in idxs]

                def m_body(mb, accs):
                    off = mb * L
                    txv = tx_s[pl.ds(off, L)]
                    tyv = ty_s[pl.ds(off, L)]
                    tzv = tz_s[pl.ds(off, L)]
                    t2v = t2_s[pl.ds(off, L)]
                    out = []
                    for j in range(SRCS):
                        val = ((t2v + bx[j] * txv)
                               + (by[j] * tyv + bz[j] * tzv))
                        out.append(jnp.minimum(accs[j], val))
                    return tuple(out)

                accs = lax.fori_loop(
                    0, M_VECS, m_body,
                    tuple(jnp.full((L,), inf) for _ in range(SRCS)))

                for j in range(SRCS):
                    i = idxs[j]
                    total = (total + wv[i] * hmin(accs[j]) + ws2v[i])
            return total

        total = lax.fori_loop(0, SC_CHUNK // L, pass_body,
                              jnp.float32(0.0))

        o_s[...] = jnp.where(lane == 0, total, jnp.float32(0.0))
        pltpu.sync_copy(o_s, out_h.at[wid])

    return k(sx, sy, sz, tx, ty, tz, w)


def _tc_block(sq_ref, rq_ref, t2_ref, w_ref, ws2_ref, o_ref):
    sq = sq_ref[0]
    rq = rq_ref[0]
    t2 = t2_ref[0]
    rowmin = None
    for mc in range(M // MT):
        cross = lax.dot_general(
            sq, rq[:, mc * MT:(mc + 1) * MT], (((1,), (0,)), ((), ())),
            preferred_element_type=jnp.float32)
        val = t2[:, mc * MT:(mc + 1) * MT] + cross
        cmin = jnp.min(val, axis=1, keepdims=True)
        rowmin = cmin if rowmin is None else jnp.minimum(rowmin, cmin)
    part = jnp.sum(w_ref[0] * rowmin) + jnp.sum(ws2_ref[0])
    row = lax.broadcasted_iota(jnp.int32, (8, 128), 0)
    col = lax.broadcasted_iota(jnp.int32, (8, 128), 1)
    o_ref[0, 0] = jnp.where((row == 0) & (col == 0), part, 0.0)


def _tc_chamfer(sq, rq, t2, w3, ws2):
    out = pl.pallas_call(
        _tc_block,
        grid=(B, NB),
        in_specs=[
            pl.BlockSpec((1, TCN, 3), lambda b, n: (b, n, 0)),
            pl.BlockSpec((1, 3, M), lambda b, n: (b, 0, 0)),
            pl.BlockSpec((1, 1, M), lambda b, n: (b, 0, 0)),
            pl.BlockSpec((1, TCN, 1), lambda b, n: (b, n, 0)),
            pl.BlockSpec((1, TCN, 1), lambda b, n: (b, n, 0)),
        ],
        out_specs=pl.BlockSpec((1, 1, 8, 128), lambda b, n: (b, n, 0, 0)),
        out_shape=jax.ShapeDtypeStruct((B, NB, 8, 128), jnp.float32),
    )(sq, rq, t2, w3, ws2)
    return jnp.sum(out)


@jax.jit
def kernel(source_cloud, target_cloud, weights_source):
    sc_part = _sc_chamfer(
        source_cloud[:, :, 0].reshape(-1),
        source_cloud[:, :, 1].reshape(-1),
        source_cloud[:, :, 2].reshape(-1),
        target_cloud[:, :, 0].reshape(-1),
        target_cloud[:, :, 1].reshape(-1),
        target_cloud[:, :, 2].reshape(-1),
        weights_source.reshape(-1))

    sqc = source_cloud.astype(jnp.bfloat16)
    rq = jnp.swapaxes(-2.0 * target_cloud.astype(jnp.bfloat16),
                      1, 2)
    t2 = jnp.sum(target_cloud * target_cloud,
                 axis=2)[:, None, :]
    s2 = jnp.sum(source_cloud * source_cloud, axis=2)
    w3 = weights_source[:, :, None]
    ws2 = (weights_source * s2)[:, :, None]

    tc_part = _tc_chamfer(sqc, rq, t2, w3, ws2)

    return (tc_part + jnp.sum(sc_part)) / B


# speedup vs baseline: 1.1647x; 1.1647x over previous
import functools

import jax
import jax.numpy as jnp
from jax import lax
from jax.experimental import pallas as pl
from jax.experimental.pallas import tpu as pltpu
from jax.experimental.pallas import tpu_sc as plsc

L = 16
NW = 32
B, N, M = 4, 4096, 4096
W_PER_B = NW // B
SC_CHUNK = 64
SC_TAIL = SC_CHUNK * W_PER_B
N_TC = N - SC_TAIL
SRCS = 8
M_VECS = M // L
TCN = 512
NB = N_TC // TCN
MT = 1024


def _sc_chamfer(sx, sy, sz, tx, ty, tz, w):
    mesh = plsc.VectorSubcoreMesh(core_axis_name="c", subcore_axis_name="s")

    @functools.partial(
        pl.kernel,
        mesh=mesh,
        out_type=jax.ShapeDtypeStruct((NW, L), jnp.float32),
        scratch_types=[
            pltpu.VMEM((SC_CHUNK,), jnp.float32),
            pltpu.VMEM((SC_CHUNK,), jnp.float32),
            pltpu.VMEM((SC_CHUNK,), jnp.float32),
            pltpu.VMEM((SC_CHUNK,), jnp.float32),
            pltpu.VMEM((M,), jnp.float32),
            pltpu.VMEM((M,), jnp.float32),
            pltpu.VMEM((M,), jnp.float32),
            pltpu.VMEM((M,), jnp.float32),
            pltpu.VMEM((L,), jnp.float32),
        ],
    )
    def k(sx_h, sy_h, sz_h, tx_h, ty_h, tz_h, w_h, out_h,
          sx_s, sy_s, sz_s, w_s, tx_s, ty_s, tz_s, t2_s, o_s):
        wid = lax.axis_index("s") * 2 + lax.axis_index("c")
        b = wid // W_PER_B
        src_base = b * N + N_TC + (wid % W_PER_B) * SC_CHUNK
        tgt_base = b * M

        pltpu.sync_copy(sx_h.at[pl.ds(src_base, SC_CHUNK)], sx_s)
        pltpu.sync_copy(sy_h.at[pl.ds(src_base, SC_CHUNK)], sy_s)
        pltpu.sync_copy(sz_h.at[pl.ds(src_base, SC_CHUNK)], sz_s)
        pltpu.sync_copy(w_h.at[pl.ds(src_base, SC_CHUNK)], w_s)
        pltpu.sync_copy(tx_h.at[pl.ds(tgt_base, M)], tx_s)
        pltpu.sync_copy(ty_h.at[pl.ds(tgt_base, M)], ty_s)
        pltpu.sync_copy(tz_h.at[pl.ds(tgt_base, M)], tz_s)

        def bq(v):
            u = lax.bitcast_convert_type(v, jnp.uint32)
            r = ((u + ((u >> 16) & jnp.uint32(1)) + jnp.uint32(0x7FFF))
                 & jnp.uint32(0xFFFF0000))
            return lax.bitcast_convert_type(r, jnp.float32)

        lane = lax.broadcasted_iota(jnp.int32, (L,), 0)

        def t2_body(i, _):
            txv = tx_s[pl.ds(i * L, L)]
            tyv = ty_s[pl.ds(i * L, L)]
            tzv = tz_s[pl.ds(i * L, L)]
            t2_s[pl.ds(i * L, L)] = txv * txv + tyv * tyv + tzv * tzv
            tx_s[pl.ds(i * L, L)] = bq(txv)
            ty_s[pl.ds(i * L, L)] = bq(tyv)
            tz_s[pl.ds(i * L, L)] = bq(tzv)
            return 0
        lax.fori_loop(0, M_VECS, t2_body, 0)

        inf = jnp.float32(jnp.inf)

        def hmin(v):
            for sh in (8, 4, 2, 1):
                v = jnp.minimum(
                    v, v.at[lane ^ sh].get(mode="promise_in_bounds"))
            return v[0]

        def pass_body(p, total):
            base = p * L
            sxv = sx_s[pl.ds(base, L)]
            syv = sy_s[pl.ds(base, L)]
            szv = sz_s[pl.ds(base, L)]
            wv = w_s[pl.ds(base, L)]
            s2v = sxv * sxv + syv * syv + szv * szv
            ws2v = wv * s2v
            sxq = bq(sxv)
            syq = bq(syv)
            szq = bq(szv)

            for half in range(L // SRCS):
                idxs = [half * SRCS + j for j in range(SRCS)]
                bx = [jnp.broadcast_to(-2.0 * sxq[i], (L,)) for i in idxs]
                by = [jnp.broadcast_to(-2.0 * syq[i], (L,)) for i in idxs]
                bz = [jnp.broadcast_to(-2.0 * szq[i], (L,)) for i in idxs]

                def m_body(mb, accs):
                    off = mb * L
                    txv = tx_s[pl.ds(off, L)]
                    tyv = ty_s[pl.ds(off, L)]
                    tzv = tz_s[pl.ds(off, L)]
                    t2v = t2_s[pl.ds(off, L)]
                    out = []
                    for j in range(SRCS):
                        val = ((t2v + bx[j] * txv)
                               + (by[j] * tyv + bz[j] * tzv))
                        out.append(jnp.minimum(accs[j], val))
                    return tuple(out)

                accs = lax.fori_loop(
                    0, M_VECS, m_body,
                    tuple(jnp.full((L,), inf) for _ in range(SRCS)))

                for j in range(SRCS):
                    i = idxs[j]
                    total = (total + wv[i] * hmin(accs[j]) + ws2v[i])
            return total

        total = lax.fori_loop(0, SC_CHUNK // L, pass_body,
                              jnp.float32(0.0))

        o_s[...] = jnp.where(lane == 0, total, jnp.float32(0.0))
        pltpu.sync_copy(o_s, out_h.at[wid])

    return k(sx, sy, sz, tx, ty, tz, w)


def _tc_block(sq_ref, rq_ref, t2_ref, w_ref, ws2_ref, o_ref):
    sq = sq_ref[0]
    rq = rq_ref[0]
    t2 = t2_ref[0]
    rowmin = None
    for mc in range(M // MT):
        cross = lax.dot_general(
            sq, rq[:, mc * MT:(mc + 1) * MT], (((0,), (0,)), ((), ())),
            preferred_element_type=jnp.float32)
        val = t2[:, mc * MT:(mc + 1) * MT] + cross
        cmin = jnp.min(val, axis=1, keepdims=True)
        rowmin = cmin if rowmin is None else jnp.minimum(rowmin, cmin)
    rowmin_t = jnp.swapaxes(rowmin, 0, 1)
    part = jnp.sum(w_ref[0] * rowmin_t) + jnp.sum(ws2_ref[0])
    row = lax.broadcasted_iota(jnp.int32, (8, 128), 0)
    col = lax.broadcasted_iota(jnp.int32, (8, 128), 1)
    o_ref[0, 0] = jnp.where((row == 0) & (col == 0), part, 0.0)


def _tc_chamfer(sq, rq, t2, w2, ws2):
    out = pl.pallas_call(
        _tc_block,
        grid=(B, NB),
        in_specs=[
            pl.BlockSpec((1, 3, TCN), lambda b, n: (b, 0, n)),
            pl.BlockSpec((1, 3, M), lambda b, n: (b, 0, 0)),
            pl.BlockSpec((1, 1, M), lambda b, n: (b, 0, 0)),
            pl.BlockSpec((1, 1, TCN), lambda b, n: (b, 0, n)),
            pl.BlockSpec((1, 1, TCN), lambda b, n: (b, 0, n)),
        ],
        out_specs=pl.BlockSpec((1, 1, 8, 128), lambda b, n: (b, n, 0, 0)),
        out_shape=jax.ShapeDtypeStruct((B, NB, 8, 128), jnp.float32),
    )(sq, rq, t2, w2, ws2)
    return jnp.sum(out)


@jax.jit
def kernel(source_cloud, target_cloud, weights_source):
    sxp = source_cloud[:, :, 0]
    syp = source_cloud[:, :, 1]
    szp = source_cloud[:, :, 2]
    txp = target_cloud[:, :, 0]
    typ = target_cloud[:, :, 1]
    tzp = target_cloud[:, :, 2]

    sc_part = _sc_chamfer(
        sxp.reshape(-1), syp.reshape(-1), szp.reshape(-1),
        txp.reshape(-1), typ.reshape(-1), tzp.reshape(-1),
        weights_source.reshape(-1))

    sq = jnp.stack([sxp, syp, szp], axis=1).astype(jnp.bfloat16)
    rq = -2.0 * jnp.stack([txp, typ, tzp], axis=1).astype(jnp.bfloat16)
    t2 = (txp * txp + typ * typ + tzp * tzp)[:, None, :]
    s2 = sxp * sxp + syp * syp + szp * szp
    w2 = weights_source[:, None, :]
    ws2 = (weights_source * s2)[:, None, :]

    tc_part = _tc_chamfer(sq, rq, t2, w2, ws2)

    return (tc_part + jnp.sum(sc_part)) / B


# speedup vs baseline: 1.1679x; 1.0027x over previous
import functools

import jax
import jax.numpy as jnp
from jax import lax
from jax.experimental import pallas as pl
from jax.experimental.pallas import tpu as pltpu
from jax.experimental.pallas import tpu_sc as plsc

L = 16
NW = 32
B, N, M = 4, 4096, 4096
W_PER_B = NW // B
SC_CHUNK = 64
SC_TAIL = SC_CHUNK * W_PER_B
N_TC = N - SC_TAIL
SRCS = 8
M_VECS = M // L
TCN = 512
NB = N_TC // TCN
MT = 512


def _sc_chamfer(sx, sy, sz, tx, ty, tz, w):
    mesh = plsc.VectorSubcoreMesh(core_axis_name="c", subcore_axis_name="s")

    @functools.partial(
        pl.kernel,
        mesh=mesh,
        out_type=jax.ShapeDtypeStruct((NW, L), jnp.float32),
        scratch_types=[
            pltpu.VMEM((SC_CHUNK,), jnp.float32),
            pltpu.VMEM((SC_CHUNK,), jnp.float32),
            pltpu.VMEM((SC_CHUNK,), jnp.float32),
            pltpu.VMEM((SC_CHUNK,), jnp.float32),
            pltpu.VMEM((M,), jnp.float32),
            pltpu.VMEM((M,), jnp.float32),
            pltpu.VMEM((M,), jnp.float32),
            pltpu.VMEM((M,), jnp.float32),
            pltpu.VMEM((L,), jnp.float32),
        ],
    )
    def k(sx_h, sy_h, sz_h, tx_h, ty_h, tz_h, w_h, out_h,
          sx_s, sy_s, sz_s, w_s, tx_s, ty_s, tz_s, t2_s, o_s):
        wid = lax.axis_index("s") * 2 + lax.axis_index("c")
        b = wid // W_PER_B
        src_base = b * N + N_TC + (wid % W_PER_B) * SC_CHUNK
        tgt_base = b * M

        pltpu.sync_copy(sx_h.at[pl.ds(src_base, SC_CHUNK)], sx_s)
        pltpu.sync_copy(sy_h.at[pl.ds(src_base, SC_CHUNK)], sy_s)
        pltpu.sync_copy(sz_h.at[pl.ds(src_base, SC_CHUNK)], sz_s)
        pltpu.sync_copy(w_h.at[pl.ds(src_base, SC_CHUNK)], w_s)
        pltpu.sync_copy(tx_h.at[pl.ds(tgt_base, M)], tx_s)
        pltpu.sync_copy(ty_h.at[pl.ds(tgt_base, M)], ty_s)
        pltpu.sync_copy(tz_h.at[pl.ds(tgt_base, M)], tz_s)

        def bq(v):
            u = lax.bitcast_convert_type(v, jnp.uint32)
            r = ((u + ((u >> 16) & jnp.uint32(1)) + jnp.uint32(0x7FFF))
                 & jnp.uint32(0xFFFF0000))
            return lax.bitcast_convert_type(r, jnp.float32)

        lane = lax.broadcasted_iota(jnp.int32, (L,), 0)

        def t2_body(i, _):
            txv = tx_s[pl.ds(i * L, L)]
            tyv = ty_s[pl.ds(i * L, L)]
            tzv = tz_s[pl.ds(i * L, L)]
            t2_s[pl.ds(i * L, L)] = txv * txv + tyv * tyv + tzv * tzv
            tx_s[pl.ds(i * L, L)] = bq(txv)
            ty_s[pl.ds(i * L, L)] = bq(tyv)
            tz_s[pl.ds(i * L, L)] = bq(tzv)
            return 0
        lax.fori_loop(0, M_VECS, t2_body, 0)

        inf = jnp.float32(jnp.inf)

        def hmin(v):
            for sh in (8, 4, 2, 1):
                v = jnp.minimum(
                    v, v.at[lane ^ sh].get(mode="promise_in_bounds"))
            return v[0]

        def pass_body(p, total):
            base = p * L
            sxv = sx_s[pl.ds(base, L)]
            syv = sy_s[pl.ds(base, L)]
            szv = sz_s[pl.ds(base, L)]
            wv = w_s[pl.ds(base, L)]
            s2v = sxv * sxv + syv * syv + szv * szv
            ws2v = wv * s2v
            sxq = bq(sxv)
            syq = bq(syv)
            szq = bq(szv)

            for half in range(L // SRCS):
                idxs = [half * SRCS + j for j in range(SRCS)]
                bx = [jnp.broadcast_to(-2.0 * sxq[i], (L,)) for i in idxs]
                by = [jnp.broadcast_to(-2.0 * syq[i], (L,)) for i in idxs]
                bz = [jnp.broadcast_to(-2.0 * szq[i], (L,)) for i in idxs]

                def m_body(mb, accs):
                    off = mb * L
                    txv = tx_s[pl.ds(off, L)]
                    tyv = ty_s[pl.ds(off, L)]
                    tzv = tz_s[pl.ds(off, L)]
                    t2v = t2_s[pl.ds(off, L)]
                    out = []
                    for j in range(SRCS):
                        val = ((t2v + bx[j] * txv)
                               + (by[j] * tyv + bz[j] * tzv))
                        out.append(jnp.minimum(accs[j], val))
                    return tuple(out)

                accs = lax.fori_loop(
                    0, M_VECS, m_body,
                    tuple(jnp.full((L,), inf) for _ in range(SRCS)))

                for j in range(SRCS):
                    i = idxs[j]
                    total = (total + wv[i] * hmin(accs[j]) + ws2v[i])
            return total

        total = lax.fori_loop(0, SC_CHUNK // L, pass_body,
                              jnp.float32(0.0))

        o_s[...] = jnp.where(lane == 0, total, jnp.float32(0.0))
        pltpu.sync_copy(o_s, out_h.at[wid])

    return k(sx, sy, sz, tx, ty, tz, w)


def _tc_block(sq_ref, rq_ref, t2_ref, w_ref, ws2_ref, o_ref):
    sq = sq_ref[0]
    rq = rq_ref[0]
    t2 = t2_ref[0]
    rowmin = None
    for mc in range(M // MT):
        cross = lax.dot_general(
            sq, rq[:, mc * MT:(mc + 1) * MT], (((0,), (0,)), ((), ())),
            preferred_element_type=jnp.float32)
        val = t2[:, mc * MT:(mc + 1) * MT] + cross
        cmin = jnp.min(val, axis=1, keepdims=True)
        rowmin = cmin if rowmin is None else jnp.minimum(rowmin, cmin)
    rowmin_t = jnp.swapaxes(rowmin, 0, 1)
    part = jnp.sum(w_ref[0] * rowmin_t) + jnp.sum(ws2_ref[0])
    row = lax.broadcasted_iota(jnp.int32, (8, 128), 0)
    col = lax.broadcasted_iota(jnp.int32, (8, 128), 1)
    o_ref[0, 0] = jnp.where((row == 0) & (col == 0), part, 0.0)


def _tc_chamfer(sq, rq, t2, w2, ws2):
    out = pl.pallas_call(
        _tc_block,
        grid=(B, NB),
        in_specs=[
            pl.BlockSpec((1, 3, TCN), lambda b, n: (b, 0, n)),
            pl.BlockSpec((1, 3, M), lambda b, n: (b, 0, 0)),
            pl.BlockSpec((1, 1, M), lambda b, n: (b, 0, 0)),
            pl.BlockSpec((1, 1, TCN), lambda b, n: (b, 0, n)),
            pl.BlockSpec((1, 1, TCN), lambda b, n: (b, 0, n)),
        ],
        out_specs=pl.BlockSpec((1, 1, 8, 128), lambda b, n: (b, n, 0, 0)),
        out_shape=jax.ShapeDtypeStruct((B, NB, 8, 128), jnp.float32),
    )(sq, rq, t2, w2, ws2)
    return jnp.sum(out)


@jax.jit
def kernel(source_cloud, target_cloud, weights_source):
    sxp = source_cloud[:, :, 0]
    syp = source_cloud[:, :, 1]
    szp = source_cloud[:, :, 2]
    txp = target_cloud[:, :, 0]
    typ = target_cloud[:, :, 1]
    tzp = target_cloud[:, :, 2]

    sc_part = _sc_chamfer(
        sxp.reshape(-1), syp.reshape(-1), szp.reshape(-1),
        txp.reshape(-1), typ.reshape(-1), tzp.reshape(-1),
        weights_source.reshape(-1))

    sq = jnp.stack([sxp, syp, szp], axis=1).astype(jnp.bfloat16)
    rq = -2.0 * jnp.stack([txp, typ, tzp], axis=1).astype(jnp.bfloat16)
    t2 = (txp * txp + typ * typ + tzp * tzp)[:, None, :]
    s2 = sxp * sxp + syp * syp + szp * szp
    w2 = weights_source[:, None, :]
    ws2 = (weights_source * s2)[:, None, :]

    tc_part = _tc_chamfer(sq, rq, t2, w2, ws2)

    return (tc_part + jnp.sum(sc_part)) / B
